# baseline (speedup 1.0000x reference)
import functools

import jax
import jax.numpy as jnp
from jax import lax
from jax.experimental import pallas as pl
from jax.experimental.pallas import tpu as pltpu
from jax.experimental.pallas import tpu_sc as plsc

NC = 2
NS = 16
NW = NC * NS
CH = 128
KC = 3
SENTINEL = 1 << 30
WBR = 224


def _stage_tc(edge_index, e_pad):
    e = edge_index.shape[1]
    r_pad = e_pad // CH
    grid = 1
    for g in range(64, 1, -1):
        if r_pad % g == 0 and (r_pad // g) % 8 == 0:
            grid = g
            break
    blkr = r_pad // grid
    blke = blkr * CH
    imax = jnp.iinfo(jnp.int32).max

    def body(x_ref, rows_ref, cols_ref, min_ref):
        i = pl.program_id(0)
        pos = lax.broadcasted_iota(jnp.int32, (blke,), 0) + i * blke
        valid = pos < e
        x0 = x_ref[0].astype(jnp.int32)
        x1 = x_ref[1].astype(jnp.int32)
        rows_ref[...] = jnp.where(valid, x0, SENTINEL)
        cols_ref[...] = jnp.where(valid, x1, (pos * 61) & 0xFFFF)

        @pl.when(i == 0)
        def _():
            min_ref[...] = jnp.full((8, 128), imax, jnp.int32)

        min_ref[...] = jnp.minimum(
            min_ref[...], jnp.min(jnp.where(valid, x0, imax)))

    return pl.pallas_call(
        body,
        grid=(grid,),
        in_specs=[pl.BlockSpec((2, blke), lambda i: (0, i))],
        out_specs=[
            pl.BlockSpec((blke,), lambda i: (i,)),
            pl.BlockSpec((blke,), lambda i: (i,)),
            pl.BlockSpec((8, 128), lambda i: (0, 0)),
        ],
        out_shape=[
            jax.ShapeDtypeStruct((e_pad,), jnp.int32),
            jax.ShapeDtypeStruct((e_pad,), jnp.int32),
            jax.ShapeDtypeStruct((8, 128), jnp.int32),
        ],
    )(edge_index)


def _combine_tc(p0, p1, bt):
    r = p0.shape[0]

    def body(a_ref, b_ref, bias_ref, o_ref):
        o_ref[...] = a_ref[...] + b_ref[...] + bias_ref[...]

    return pl.pallas_call(
        body,
        out_shape=jax.ShapeDtypeStruct((r, 128), jnp.float32),
    )(p0, p1, bt)


def _make_sc_kernel(n_nodes, c, ni, acc_rows):
    mesh = plsc.VectorSubcoreMesh(
        core_axis_name="c", subcore_axis_name="s", num_cores=NC, num_subcores=NS
    )
    NB = 3
    assert ni % NB == 0

    @functools.partial(
        pl.kernel,
        out_type=jax.ShapeDtypeStruct((NC * acc_rows * c // 128, 128),
                                      jnp.float32),
        mesh=mesh,
        scratch_types=[
            [pltpu.VMEM((KC * CH,), jnp.int32) for _ in range(NB)],
            [pltpu.VMEM((KC * CH,), jnp.int32) for _ in range(NB)],
            [pltpu.VMEM((KC, CH), jnp.int32) for _ in range(NB)],
            [pltpu.VMEM((KC, CH, c), jnp.float32) for _ in range(NB)],
            pltpu.VMEM((1, 16), jnp.int32),
            pltpu.VMEM((WBR, c), jnp.float32),
            pltpu.VMEM((WBR * c // 128, 128), jnp.float32),
            pltpu.VMEM_SHARED((acc_rows, c), jnp.float32),
            [pltpu.SemaphoreType.DMA for _ in range(NB)],
            [pltpu.SemaphoreType.DMA for _ in range(NB)],
            [pltpu.SemaphoreType.DMA for _ in range(NB)],
        ],
        compiler_params=pltpu.CompilerParams(use_tc_tiling_on_sc=False),
    )
    def sc_fn(rows_hbm, cols_hbm, wt_hbm, mvec_hbm, zeros_hbm, out_hbm,
              cols_v, rows_v, sidx_v, gbuf, mvec_v, vbuf16, vbuf128, acc,
              semI, semG, semS):
        cid = lax.axis_index("c")
        sid = lax.axis_index("s")
        wid = sid * NC + cid

        zstripe = acc_rows // NS

        def zfire(k, carry):
            pltpu.async_copy(
                zeros_hbm, acc.at[pl.ds(sid * zstripe + k * WBR, WBR)],
                semI[0])
            return carry

        def zdrain(k, carry):
            pltpu.make_async_copy(
                zeros_hbm, acc.at[pl.ds(sid * zstripe + k * WBR, WBR)],
                semI[0]).wait()
            return carry

        lax.fori_loop(0, zstripe // WBR, zfire, 0)
        pltpu.sync_copy(mvec_hbm.at[pl.ds(0, 1), pl.ds(0, 16)], mvec_v)
        lax.fori_loop(0, zstripe // WBR, zdrain, 0)
        plsc.subcore_barrier()

        mv = mvec_v[0, pl.ds(0, 16)]
        clampv = jnp.full((16,), n_nodes, jnp.int32)
        iotas = [lax.iota(jnp.int32, 16) + 16 * i for i in range(CH // 16)]
        zero16 = jnp.zeros((16,), jnp.int32)

        def fire_idx(it, s):
            eb = (it * NW + wid) * KC * CH
            pltpu.async_copy(cols_hbm.at[pl.ds(eb, KC * CH)], cols_v[s], semI[s])
            pltpu.async_copy(rows_hbm.at[pl.ds(eb, KC * CH)], rows_v[s], semI[s])

        def wait_idx(s):
            pltpu.make_async_copy(
                cols_hbm.at[pl.ds(0, KC * CH)], cols_v[s], semI[s]).wait()
            pltpu.make_async_copy(
                rows_hbm.at[pl.ds(0, KC * CH)], rows_v[s], semI[s]).wait()

        def fire_gathers(s):
            for j in range(KC):
                pltpu.async_copy(wt_hbm.at[cols_v[s].at[pl.ds(j * CH, CH)]],
                                 gbuf[s].at[j], semG[s])

        def wait_gathers(s):
            for j in range(KC):
                pltpu.make_async_copy(
                    wt_hbm.at[cols_v[s].at[pl.ds(j * CH, CH)]],
                    gbuf[s].at[j], semG[s]).wait()

        def prep(s):
            for j in range(KC):
                for i in range(CH // 16):
                    r = rows_v[s][pl.ds(j * CH + i * 16, 16)]
                    sh = jnp.minimum(r - mv, clampv)
                    sh = sh + jnp.where(sh == clampv, iotas[i], zero16)
                    sidx_v[s][j, pl.ds(i * 16, 16)] = sh

        def fire_scatters(s):
            for j in range(KC):
                pltpu.async_copy(gbuf[s].at[j], acc.at[sidx_v[s].at[j]],
                                 semS[s], add=True)

        def wait_scatters(s):
            for j in range(KC):
                pltpu.make_async_copy(
                    gbuf[s].at[j], acc.at[sidx_v[s].at[j]], semS[s]).wait()

        fire_idx(0, 0)
        fire_idx(1, 1)
        wait_idx(0)
        fire_gathers(0)
        prep(0)

        def iter_body(it3, carry):
            for ph in range(NB):
                t = it3 * NB + ph
                s, s1, s2 = ph, (ph + 1) % NB, (ph + 2) % NB

                @pl.when(t + 2 < ni)
                def _():
                    fire_idx(t + 2, s2)

                @pl.when(t + 1 < ni)
                def _():
                    wait_idx(s1)
                    fire_gathers(s1)
                    prep(s1)

                @pl.when(t > 0)
                def _():
                    wait_scatters(s2)

                wait_gathers(s)
                fire_scatters(s)
            return carry

        lax.fori_loop(0, ni // NB, iter_body, 0)
        wait_scatters(NB - 1)

        plsc.subcore_barrier()
        obase = (cid * NS + sid) * (zstripe * c // 128)
        wb8 = WBR * c // 128

        def wb_body(k, carry):
            pltpu.sync_copy(
                acc.at[pl.ds(sid * zstripe + k * WBR, WBR)], vbuf16)

            def rel_body(g, carry2):
                for u in range(128 // c):
                    v = vbuf16[g * (128 // c) + u, pl.ds(0, c)]
                    vbuf128[g, pl.ds(u * c, c)] = v
                return carry2

            lax.fori_loop(0, wb8, rel_body, 0)
            pltpu.sync_copy(vbuf128, out_hbm.at[pl.ds(obase + k * wb8, wb8)])
            return carry

        lax.fori_loop(0, zstripe // WBR, wb_body, 0)

    return sc_fn


def kernel(edge_index, W, b):
    c, n = W.shape
    e = edge_index.shape[1]

    per_tile_chunks = -(-e // (NW * CH))
    ni = -(-per_tile_chunks // KC)
    ni = -(-ni // 3) * 3
    e_pad = NW * ni * KC * CH
    rows2d, cols2d, mins = _stage_tc(edge_index, e_pad)

    acc_rows = -(-(n + CH) // (NS * 8 * 128 // c)) * (NS * 8 * 128 // c)
    arow8 = acc_rows * c // 128
    zeros = jnp.zeros((WBR, c), jnp.float32)
    wt = W.T

    sc_fn = _make_sc_kernel(n, c, ni, acc_rows)
    partials = sc_fn(rows2d, cols2d, wt, mins, zeros)

    flat = n * c // 128
    p0 = partials[:flat]
    p1 = partials[arow8:arow8 + flat]
    bt = jnp.tile(b, 128 // c).reshape(1, 128)
    out = _combine_tc(p0, p1, bt)
    return out.reshape(n, c)


# speedup vs baseline: 1.1100x; 1.1100x over previous
import functools

import jax
import jax.numpy as jnp
from jax import lax
from jax.experimental import pallas as pl
from jax.experimental.pallas import tpu as pltpu
from jax.experimental.pallas import tpu_sc as plsc

NC = 2
NS = 16
NW = NC * NS
CH = 128
KC = 4
SENTINEL = 1 << 30
WBR = 56


def _stage_tc(edge_index, e_pad):
    e = edge_index.shape[1]
    r_pad = e_pad // CH
    grid = 1
    for g in range(64, 1, -1):
        if r_pad % g == 0 and (r_pad // g) % 8 == 0:
            grid = g
            break
    blkr = r_pad // grid
    blke = blkr * CH
    imax = jnp.iinfo(jnp.int32).max

    def body(x_ref, rows_ref, cols_ref, min_ref):
        i = pl.program_id(0)
        sub = lax.broadcasted_iota(jnp.int32, (blkr, CH), 0)
        lane = lax.broadcasted_iota(jnp.int32, (blkr, CH), 1)
        pos = (i * blkr + sub) * CH + lane
        valid = pos < e
        x0 = x_ref[0].reshape(blkr, CH).astype(jnp.int32)
        x1 = x_ref[1].reshape(blkr, CH).astype(jnp.int32)
        rows_ref[...] = jnp.where(valid, x0, SENTINEL)
        cols_ref[...] = jnp.where(valid, x1, (pos * 61) & 0xFFFF)

        @pl.when(i == 0)
        def _():
            min_ref[...] = jnp.full((8, 128), imax, jnp.int32)

        min_ref[...] = jnp.minimum(
            min_ref[...], jnp.min(jnp.where(valid, x0, imax)))

    return pl.pallas_call(
        body,
        grid=(grid,),
        in_specs=[pl.BlockSpec((2, blke), lambda i: (0, i))],
        out_specs=[
            pl.BlockSpec((blkr, CH), lambda i: (i, 0)),
            pl.BlockSpec((blkr, CH), lambda i: (i, 0)),
            pl.BlockSpec((8, 128), lambda i: (0, 0)),
        ],
        out_shape=[
            jax.ShapeDtypeStruct((r_pad, CH), jnp.int32),
            jax.ShapeDtypeStruct((r_pad, CH), jnp.int32),
            jax.ShapeDtypeStruct((8, 128), jnp.int32),
        ],
    )(edge_index)


def _combine_tc(p0, p1, bt):
    r = p0.shape[0]

    def body(a_ref, b_ref, bias_ref, o_ref):
        o_ref[...] = a_ref[...] + b_ref[...] + bias_ref[...]

    return pl.pallas_call(
        body,
        out_shape=jax.ShapeDtypeStruct((r, 128), jnp.float32),
    )(p0, p1, bt)


def _make_sc_kernel(n_nodes, c, ni, acc_rows):
    mesh = plsc.VectorSubcoreMesh(
        core_axis_name="c", subcore_axis_name="s", num_cores=NC, num_subcores=NS
    )
    NB = 3
    assert ni % NB == 0

    @functools.partial(
        pl.kernel,
        out_type=jax.ShapeDtypeStruct((NC * acc_rows * c // 128, 128),
                                      jnp.float32),
        mesh=mesh,
        scratch_types=[
            [pltpu.VMEM((KC, CH), jnp.int32) for _ in range(NB)],
            [pltpu.VMEM((KC, CH), jnp.int32) for _ in range(NB)],
            [pltpu.VMEM((KC, CH, c), jnp.float32) for _ in range(NB)],
            pltpu.VMEM((1, 16), jnp.int32),
            pltpu.VMEM((WBR, c), jnp.float32),
            pltpu.VMEM((WBR * c // 128, 128), jnp.float32),
            pltpu.VMEM_SHARED((acc_rows, c), jnp.float32),
            [pltpu.SemaphoreType.DMA for _ in range(NB)],
            [pltpu.SemaphoreType.DMA for _ in range(NB)],
            [pltpu.SemaphoreType.DMA for _ in range(NB)],
        ],
        compiler_params=pltpu.CompilerParams(use_tc_tiling_on_sc=False),
    )
    def sc_fn(rows_hbm, cols_hbm, wt_hbm, mvec_hbm, zeros_hbm, out_hbm,
              cols_v, rows_v, gbuf, mvec_v, vbuf16, vbuf128, acc,
              semI, semG, semS):
        cid = lax.axis_index("c")
        sid = lax.axis_index("s")
        wid = sid * NC + cid

        zstripe = acc_rows // NS

        pltpu.sync_copy(zeros_hbm, acc.at[pl.ds(sid * zstripe, zstripe)])
        pltpu.sync_copy(mvec_hbm.at[pl.ds(0, 1), pl.ds(0, 16)], mvec_v)
        plsc.subcore_barrier()

        mv = mvec_v[0, pl.ds(0, 16)]
        clampv = jnp.full((16,), n_nodes, jnp.int32)
        iotas = [lax.iota(jnp.int32, 16) + 16 * i for i in range(CH // 16)]
        zero16 = jnp.zeros((16,), jnp.int32)

        def fire_idx(it, s):
            cb = (it * NW + wid) * KC
            pltpu.async_copy(cols_hbm.at[pl.ds(cb, KC)], cols_v[s], semI[s])
            pltpu.async_copy(rows_hbm.at[pl.ds(cb, KC)], rows_v[s], semI[s])

        def wait_idx(s):
            pltpu.make_async_copy(
                cols_hbm.at[pl.ds(0, KC)], cols_v[s], semI[s]).wait()
            pltpu.make_async_copy(
                rows_hbm.at[pl.ds(0, KC)], rows_v[s], semI[s]).wait()

        def fire_gathers(s):
            for j in range(KC):
                pltpu.async_copy(wt_hbm.at[cols_v[s].at[j]], gbuf[s].at[j],
                                 semG[s])

        def wait_gathers(s):
            for j in range(KC):
                pltpu.make_async_copy(
                    wt_hbm.at[cols_v[s].at[j]], gbuf[s].at[j], semG[s]).wait()

        def prep(s):
            for j in range(KC):
                for i in range(CH // 16):
                    r = rows_v[s][j, pl.ds(i * 16, 16)]
                    sh = jnp.minimum(r - mv, clampv)
                    sh = sh + jnp.where(sh == clampv, iotas[i], zero16)
                    rows_v[s][j, pl.ds(i * 16, 16)] = sh

        def fire_scatters(s):
            for j in range(KC):
                pltpu.async_copy(gbuf[s].at[j], acc.at[rows_v[s].at[j]],
                                 semS[s], add=True)

        def wait_scatters(s):
            for j in range(KC):
                pltpu.make_async_copy(
                    gbuf[s].at[j], acc.at[rows_v[s].at[j]], semS[s]).wait()

        fire_idx(0, 0)
        fire_idx(1, 1)
        wait_idx(0)
        fire_gathers(0)
        prep(0)

        def iter_body(it3, carry):
            for ph in range(NB):
                t = it3 * NB + ph
                s, s1, s2 = ph, (ph + 1) % NB, (ph + 2) % NB

                @pl.when(t > 0)
                def _():
                    wait_scatters(s2)

                @pl.when(t + 2 < ni)
                def _():
                    fire_idx(t + 2, s2)

                @pl.when(t + 1 < ni)
                def _():
                    wait_idx(s1)
                    fire_gathers(s1)
                    prep(s1)

                wait_gathers(s)
                fire_scatters(s)
            return carry

        lax.fori_loop(0, ni // NB, iter_body, 0)
        wait_scatters(NB - 1)

        plsc.subcore_barrier()
        obase = (cid * NS + sid) * (zstripe * c // 128)
        wb8 = WBR * c // 128

        def wb_body(k, carry):
            pltpu.sync_copy(
                acc.at[pl.ds(sid * zstripe + k * WBR, WBR)], vbuf16)

            def rel_body(g, carry2):
                for u in range(128 // c):
                    v = vbuf16[g * (128 // c) + u, pl.ds(0, c)]
                    vbuf128[g, pl.ds(u * c, c)] = v
                return carry2

            lax.fori_loop(0, wb8, rel_body, 0)
            pltpu.sync_copy(vbuf128, out_hbm.at[pl.ds(obase + k * wb8, wb8)])
            return carry

        lax.fori_loop(0, zstripe // WBR, wb_body, 0)

    return sc_fn


def kernel(edge_index, W, b):
    c, n = W.shape
    e = edge_index.shape[1]

    per_tile_chunks = -(-e // (NW * CH))
    ni = -(-per_tile_chunks // KC)
    ni = -(-ni // 3) * 3
    e_pad = NW * ni * KC * CH
    rows2d, cols2d, mins = _stage_tc(edge_index, e_pad)

    acc_rows = -(-(n + CH) // (NS * 8 * 128 // c)) * (NS * 8 * 128 // c)
    arow8 = acc_rows * c // 128
    zeros = jnp.zeros((acc_rows // NS, c), jnp.float32)
    wt = W.T

    sc_fn = _make_sc_kernel(n, c, ni, acc_rows)
    partials = sc_fn(rows2d, cols2d, wt, mins, zeros)

    flat = n * c // 128
    p0 = partials[:flat]
    p1 = partials[arow8:arow8 + flat]
    bt = jnp.tile(b, 128 // c).reshape(1, 128)
    out = _combine_tc(p0, p1, bt)
    return out.reshape(n, c)


# speedup vs baseline: 1.1768x; 1.0602x over previous
import functools

import jax
import jax.numpy as jnp
from jax import lax
from jax.experimental import pallas as pl
from jax.experimental.pallas import tpu as pltpu
from jax.experimental.pallas import tpu_sc as plsc

NC = 2
NS = 16
NW = NC * NS
CH = 128
KC = 3
SENTINEL = 1 << 30
WBR = 112


def _stage_tc(edge_index, e_pad):
    e = edge_index.shape[1]
    r_pad = e_pad // CH
    grid = 1
    for g in range(64, 1, -1):
        if r_pad % g == 0 and (r_pad // g) % 8 == 0:
            grid = g
            break
    blkr = r_pad // grid
    blke = blkr * CH
    imax = jnp.iinfo(jnp.int32).max

    def body(x_ref, rows_ref, cols_ref, min_ref):
        i = pl.program_id(0)
        sub = lax.broadcasted_iota(jnp.int32, (blkr, CH), 0)
        lane = lax.broadcasted_iota(jnp.int32, (blkr, CH), 1)
        pos = (i * blkr + sub) * CH + lane
        valid = pos < e
        x0 = x_ref[0].reshape(blkr, CH).astype(jnp.int32)
        x1 = x_ref[1].reshape(blkr, CH).astype(jnp.int32)
        rows_ref[...] = jnp.where(valid, x0, SENTINEL)
        cols_ref[...] = jnp.where(valid, x1, (pos * 61) & 0xFFFF)

        @pl.when(i == 0)
        def _():
            min_ref[...] = jnp.full((8, 128), imax, jnp.int32)

        min_ref[...] = jnp.minimum(
            min_ref[...], jnp.min(jnp.where(valid, x0, imax)))

    return pl.pallas_call(
        body,
        grid=(grid,),
        in_specs=[pl.BlockSpec((2, blke), lambda i: (0, i))],
        out_specs=[
            pl.BlockSpec((blkr, CH), lambda i: (i, 0)),
            pl.BlockSpec((blkr, CH), lambda i: (i, 0)),
            pl.BlockSpec((8, 128), lambda i: (0, 0)),
        ],
        out_shape=[
            jax.ShapeDtypeStruct((r_pad, CH), jnp.int32),
            jax.ShapeDtypeStruct((r_pad, CH), jnp.int32),
            jax.ShapeDtypeStruct((8, 128), jnp.int32),
        ],
    )(edge_index)


def _combine_tc(p0, p1, bt):
    r = p0.shape[0]

    def body(a_ref, b_ref, bias_ref, o_ref):
        o_ref[...] = a_ref[...] + b_ref[...] + bias_ref[...]

    return pl.pallas_call(
        body,
        out_shape=jax.ShapeDtypeStruct((r, 128), jnp.float32),
    )(p0, p1, bt)


def _make_sc_kernel(n_nodes, c, ni, acc_rows):
    mesh = plsc.VectorSubcoreMesh(
        core_axis_name="c", subcore_axis_name="s", num_cores=NC, num_subcores=NS
    )
    NB = 3
    assert ni % NB == 0

    @functools.partial(
        pl.kernel,
        out_type=jax.ShapeDtypeStruct((NC * acc_rows * c // 128, 128),
                                      jnp.float32),
        mesh=mesh,
        scratch_types=[
            [pltpu.VMEM((KC, CH), jnp.int32) for _ in range(NB)],
            [pltpu.VMEM((KC, CH), jnp.int32) for _ in range(NB)],
            [pltpu.VMEM((KC, CH), jnp.int32) for _ in range(NB)],
            [pltpu.VMEM((KC, CH, c), jnp.float32) for _ in range(NB)],
            pltpu.VMEM((1, 16), jnp.int32),
            [pltpu.VMEM((WBR, c), jnp.float32) for _ in range(2)],
            [pltpu.VMEM((WBR * c // 128, 128), jnp.float32)
             for _ in range(2)],
            pltpu.VMEM_SHARED((acc_rows, c), jnp.float32),
            [pltpu.SemaphoreType.DMA for _ in range(NB)],
            [pltpu.SemaphoreType.DMA for _ in range(NB)],
            [pltpu.SemaphoreType.DMA for _ in range(NB)],
        ],
        compiler_params=pltpu.CompilerParams(use_tc_tiling_on_sc=False),
    )
    def sc_fn(rows_hbm, cols_hbm, wt_hbm, mvec_hbm, zeros_hbm, out_hbm,
              cols_v, rows_v, sidx_v, gbuf, mvec_v, vbuf16, vbuf128, acc,
              semI, semG, semS):
        cid = lax.axis_index("c")
        sid = lax.axis_index("s")
        wid = sid * NC + cid

        zstripe = acc_rows // NS
        clampv = jnp.full((16,), n_nodes, jnp.int32)
        iotas = [lax.iota(jnp.int32, 16) + 16 * i for i in range(CH // 16)]
        zero16 = jnp.zeros((16,), jnp.int32)

        def fire_idx(it, s):
            cb = (it * NW + wid) * KC
            pltpu.async_copy(cols_hbm.at[pl.ds(cb, KC)], cols_v[s], semI[s])
            pltpu.async_copy(rows_hbm.at[pl.ds(cb, KC)], rows_v[s], semI[s])

        def wait_idx(s):
            pltpu.make_async_copy(
                cols_hbm.at[pl.ds(0, KC)], cols_v[s], semI[s]).wait()
            pltpu.make_async_copy(
                rows_hbm.at[pl.ds(0, KC)], rows_v[s], semI[s]).wait()

        def fire_gathers(s):
            for j in range(KC):
                pltpu.async_copy(wt_hbm.at[cols_v[s].at[j]], gbuf[s].at[j],
                                 semG[s])

        def wait_gathers(s):
            for j in range(KC):
                pltpu.make_async_copy(
                    wt_hbm.at[cols_v[s].at[j]], gbuf[s].at[j], semG[s]).wait()

        def prep(s):
            for j in range(KC):
                for i in range(CH // 16):
                    r = rows_v[s][j, pl.ds(i * 16, 16)]
                    sh = jnp.minimum(r - mv, clampv)
                    sh = sh + jnp.where(sh == clampv, iotas[i], zero16)
                    sidx_v[s][j, pl.ds(i * 16, 16)] = sh

        def fire_scatters(s):
            for j in range(KC):
                pltpu.async_copy(gbuf[s].at[j], acc.at[sidx_v[s].at[j]],
                                 semS[s], add=True)

        def wait_scatters(s):
            for j in range(KC):
                pltpu.make_async_copy(
                    gbuf[s].at[j], acc.at[sidx_v[s].at[j]], semS[s]).wait()

        fire_idx(0, 0)
        fire_idx(1, 1)
        pltpu.sync_copy(zeros_hbm, acc.at[pl.ds(sid * zstripe, zstripe)])
        pltpu.sync_copy(mvec_hbm.at[pl.ds(0, 1), pl.ds(0, 16)], mvec_v)
        plsc.subcore_barrier()
        mv = mvec_v[0, pl.ds(0, 16)]
        wait_idx(0)
        fire_gathers(0)
        prep(0)

        def iter_body(it3, carry):
            for ph in range(NB):
                t = it3 * NB + ph
                s, s1, s2 = ph, (ph + 1) % NB, (ph + 2) % NB

                @pl.when(t + 2 < ni)
                def _():
                    fire_idx(t + 2, s2)

                @pl.when(t + 1 < ni)
                def _():
                    wait_idx(s1)
                    fire_gathers(s1)
                    prep(s1)

                @pl.when(t > 0)
                def _():
                    wait_scatters(s2)

                wait_gathers(s)
                fire_scatters(s)
            return carry

        lax.fori_loop(0, ni // NB, iter_body, 0)
        wait_scatters(NB - 1)

        plsc.subcore_barrier()
        obase = (cid * NS + sid) * (zstripe * c // 128)
        wb8 = WBR * c // 128
        nwb = zstripe // WBR

        def wb_in(k, d):
            pltpu.async_copy(
                acc.at[pl.ds(sid * zstripe + k * WBR, WBR)], vbuf16[d],
                semI[d])

        def wb_in_wait(d):
            pltpu.make_async_copy(
                acc.at[pl.ds(0, WBR)], vbuf16[d], semI[d]).wait()

        def wb_rel(d):
            for g in range(wb8):
                for u in range(128 // c):
                    v = vbuf16[d][g * (128 // c) + u, pl.ds(0, c)]
                    vbuf128[d][g, pl.ds(u * c, c)] = v

        def wb_out(k, d):
            pltpu.async_copy(
                vbuf128[d], out_hbm.at[pl.ds(obase + k * wb8, wb8)], semG[d])

        def wb_out_wait(d):
            pltpu.make_async_copy(
                vbuf128[d], out_hbm.at[pl.ds(obase, wb8)], semG[d]).wait()

        wb_in(0, 0)

        def wb_loop(k2, carry):
            for d in range(2):
                k = k2 * 2 + d

                @pl.when(k + 1 < nwb)
                def _():
                    wb_in(k + 1, 1 - d)

                wb_in_wait(d)

                @pl.when(k > 1)
                def _():
                    wb_out_wait(d)

                wb_rel(d)
                wb_out(k, d)
            return carry

        lax.fori_loop(0, nwb // 2, wb_loop, 0)
        wb_out_wait(0)
        wb_out_wait(1)

    return sc_fn


def kernel(edge_index, W, b):
    c, n = W.shape
    e = edge_index.shape[1]

    per_tile_chunks = -(-e // (NW * CH))
    ni = -(-per_tile_chunks // KC)
    ni = -(-ni // 3) * 3
    e_pad = NW * ni * KC * CH
    rows2d, cols2d, mins = _stage_tc(edge_index, e_pad)

    acc_rows = -(-(n + CH) // (NS * 8 * 128 // c)) * (NS * 8 * 128 // c)
    arow8 = acc_rows * c // 128
    zeros = jnp.zeros((acc_rows // NS, c), jnp.float32)
    wt = W.T

    sc_fn = _make_sc_kernel(n, c, ni, acc_rows)
    partials = sc_fn(rows2d, cols2d, wt, mins, zeros)

    flat = n * c // 128
    p0 = partials[:flat]
    p1 = partials[arow8:arow8 + flat]
    bt = jnp.tile(b, 128 // c).reshape(1, 128)
    out = _combine_tc(p0, p1, bt)
    return out.reshape(n, c)
